# baseline (speedup 1.0000x reference)
import jax
import jax.numpy as jnp
from jax.experimental import pallas as pl
from jax.experimental.pallas import tpu as pltpu

B, N, M, D = 16, 2048, 2048, 3
CHUNK = 256


def _chamfer_body(pred_ref, gt_ref, out_ref):
    b = pl.program_id(0)
    pred = pred_ref[0]
    gtb = gt_ref[0]
    gtt2 = -2.0 * jax.lax.transpose(gtb, (1, 0))
    g2 = 0.25 * jnp.sum(gtt2 * gtt2, axis=0, keepdims=True)
    bmat = jnp.concatenate([gtt2, g2], axis=0)
    colmin = jnp.full((1, M), jnp.inf, dtype=jnp.float32)
    sum1 = jnp.float32(0.0)
    for c in range(N // CHUNK):
        pc = pred[c * CHUNK:(c + 1) * CHUNK, :]
        p2 = jnp.sum(pc * pc, axis=1, keepdims=True)
        ones_c = jnp.ones((CHUNK, 1), dtype=jnp.float32)
        amat = jnp.concatenate([pc, ones_c], axis=1)
        d = jax.lax.dot_general(
            amat, bmat, (((1,), (0,)), ((), ())),
            preferred_element_type=jnp.float32) + p2
        rmin = jnp.min(d, axis=1)
        sum1 = sum1 + jnp.sum(jnp.maximum(rmin, 0.0))
        colmin = jnp.minimum(colmin, jnp.min(d, axis=0, keepdims=True))
    sum2 = jnp.sum(jnp.maximum(colmin, 0.0))
    total = sum1 / (B * N) + sum2 / (B * M)

    @pl.when(b == 0)
    def _():
        out_ref[0, 0] = jnp.float32(0.0)

    out_ref[0, 0] += total


def kernel(pred, gt):
    out = pl.pallas_call(
        _chamfer_body,
        grid=(B,),
        in_specs=[
            pl.BlockSpec((1, N, D), lambda b: (b, 0, 0)),
            pl.BlockSpec((1, M, D), lambda b: (b, 0, 0)),
        ],
        out_specs=pl.BlockSpec((1, 1), lambda b: (0, 0),
                               memory_space=pltpu.SMEM),
        out_shape=jax.ShapeDtypeStruct((1, 1), jnp.float32),
        compiler_params=pltpu.CompilerParams(
            dimension_semantics=("arbitrary",)),
    )(pred, gt)
    return out[0, 0]


# speedup vs baseline: 1.2470x; 1.2470x over previous
import jax
import jax.numpy as jnp
from jax.experimental import pallas as pl
from jax.experimental.pallas import tpu as pltpu

B, N, M, D = 16, 2048, 2048, 3
CHUNK = 256


PER_STEP = 2


def _one_batch(pred, gtt2):
    g2 = 0.25 * jnp.sum(gtt2 * gtt2, axis=0, keepdims=True)
    bmat = jnp.concatenate([gtt2, g2], axis=0)
    colmin = jnp.full((1, M), jnp.inf, dtype=jnp.float32)
    sum1 = jnp.float32(0.0)
    for c in range(N // CHUNK):
        pc = pred[c * CHUNK:(c + 1) * CHUNK, :]
        p2 = jnp.sum(pc * pc, axis=1, keepdims=True)
        ones_c = jnp.ones((CHUNK, 1), dtype=jnp.float32)
        amat = jnp.concatenate([pc, ones_c], axis=1)
        d = jax.lax.dot_general(
            amat, bmat, (((1,), (0,)), ((), ())),
            preferred_element_type=jnp.float32) + p2
        rmin = jnp.min(d, axis=1)
        sum1 = sum1 + jnp.sum(jnp.maximum(rmin, 0.0))
        colmin = jnp.minimum(colmin, jnp.min(d, axis=0, keepdims=True))
    sum2 = jnp.sum(jnp.maximum(colmin, 0.0))
    return sum1 / (B * N) + sum2 / (B * M)


def _chamfer_body(pred_ref, gtt2_ref, out_ref):
    b = pl.program_id(0)
    total = jnp.float32(0.0)
    for s in range(PER_STEP):
        total = total + _one_batch(pred_ref[s], gtt2_ref[s])

    @pl.when(b == 0)
    def _():
        out_ref[0, 0] = jnp.float32(0.0)

    out_ref[0, 0] += total


def kernel(pred, gt):
    gtt2 = -2.0 * jnp.transpose(gt, (0, 2, 1))
    out = pl.pallas_call(
        _chamfer_body,
        grid=(B // PER_STEP,),
        in_specs=[
            pl.BlockSpec((PER_STEP, N, D), lambda b: (b, 0, 0)),
            pl.BlockSpec((PER_STEP, D, M), lambda b: (b, 0, 0)),
        ],
        out_specs=pl.BlockSpec((1, 1), lambda b: (0, 0),
                               memory_space=pltpu.SMEM),
        out_shape=jax.ShapeDtypeStruct((1, 1), jnp.float32),
        compiler_params=pltpu.CompilerParams(
            dimension_semantics=("arbitrary",)),
    )(pred, gtt2)
    return out[0, 0]


# speedup vs baseline: 1.2753x; 1.0227x over previous
import jax
import jax.numpy as jnp
from jax.experimental import pallas as pl
from jax.experimental.pallas import tpu as pltpu

B, N, M, D = 16, 2048, 2048, 3
CHUNK = 256


PER_STEP = 4


def _one_batch(pred, gtt2):
    g2 = 0.25 * jnp.sum(gtt2 * gtt2, axis=0, keepdims=True)
    bmat = jnp.concatenate([gtt2, g2], axis=0)
    colmin = jnp.full((1, M), jnp.inf, dtype=jnp.float32)
    sum1 = jnp.float32(0.0)
    for c in range(N // CHUNK):
        pc = pred[c * CHUNK:(c + 1) * CHUNK, :]
        p2 = jnp.sum(pc * pc, axis=1, keepdims=True)
        ones_c = jnp.ones((CHUNK, 1), dtype=jnp.float32)
        amat = jnp.concatenate([pc, ones_c], axis=1)
        d = jax.lax.dot_general(
            amat, bmat, (((1,), (0,)), ((), ())),
            preferred_element_type=jnp.float32) + p2
        rmin = jnp.min(d, axis=1)
        sum1 = sum1 + jnp.sum(jnp.maximum(rmin, 0.0))
        colmin = jnp.minimum(colmin, jnp.min(d, axis=0, keepdims=True))
    sum2 = jnp.sum(jnp.maximum(colmin, 0.0))
    return sum1 / (B * N) + sum2 / (B * M)


def _chamfer_body(pred_ref, gtt2_ref, out_ref):
    b = pl.program_id(0)
    total = jnp.float32(0.0)
    for s in range(PER_STEP):
        total = total + _one_batch(pred_ref[s], gtt2_ref[s])

    @pl.when(b == 0)
    def _():
        out_ref[0, 0] = jnp.float32(0.0)

    out_ref[0, 0] += total


def kernel(pred, gt):
    gtt2 = -2.0 * jnp.transpose(gt, (0, 2, 1))
    out = pl.pallas_call(
        _chamfer_body,
        grid=(B // PER_STEP,),
        in_specs=[
            pl.BlockSpec((PER_STEP, N, D), lambda b: (b, 0, 0)),
            pl.BlockSpec((PER_STEP, D, M), lambda b: (b, 0, 0)),
        ],
        out_specs=pl.BlockSpec((1, 1), lambda b: (0, 0),
                               memory_space=pltpu.SMEM),
        out_shape=jax.ShapeDtypeStruct((1, 1), jnp.float32),
        compiler_params=pltpu.CompilerParams(
            dimension_semantics=("arbitrary",)),
    )(pred, gtt2)
    return out[0, 0]
